# baseline (speedup 1.0000x reference)
import functools

import jax
import jax.numpy as jnp
from jax import lax
from jax.experimental import pallas as pl
from jax.experimental.pallas import tpu as pltpu
from jax.experimental.pallas import tpu_sc as plsc

_B, _C, _H, _W = 64, 384, 24, 24
_CO = _C // 2
_RTOT = _B * _H * _W
_NW = 32
_RPW = _RTOT // _NW
_R = 96
_NCH = _RPW // _R
_NQ = _CO // 16


def _make_sc_select():
    mesh = plsc.VectorSubcoreMesh(core_axis_name="c", subcore_axis_name="s")

    @functools.partial(
        pl.kernel,
        mesh=mesh,
        out_type=jax.ShapeDtypeStruct((_RTOT, _CO), jnp.float32),
        scratch_types=[
            pltpu.VMEM((_R, _C), jnp.float32),
            pltpu.VMEM((_R, _C), jnp.float32),
            pltpu.VMEM((_R, _CO), jnp.float32),
            pltpu.VMEM((_R, _CO), jnp.float32),
            pltpu.SemaphoreType.DMA,
            pltpu.SemaphoreType.DMA,
            pltpu.SemaphoreType.DMA,
            pltpu.SemaphoreType.DMA,
        ],
    )
    def sc_select(in_hbm, out_hbm, ibuf0, ibuf1, obuf0, obuf1,
                  isem0, isem1, osem0, osem1):
        wid = lax.axis_index("s") * 2 + lax.axis_index("c")
        base = wid * _RPW
        ibufs = (ibuf0, ibuf1)
        obufs = (obuf0, obuf1)
        isems = (isem0, isem1)
        osems = (osem0, osem1)

        lane = lax.iota(jnp.int32, 16)
        evens = (2 * lane) % 16
        low = lane < 8
        dnums = lax.GatherDimensionNumbers(
            offset_dims=(), collapsed_slice_dims=(0,), start_index_map=(0,))

        def compact(a, b):
            ga = lax.gather(a, evens[:, None], dnums, slice_sizes=(1,),
                            mode=lax.GatherScatterMode.PROMISE_IN_BOUNDS)
            gb = lax.gather(b, evens[:, None], dnums, slice_sizes=(1,),
                            mode=lax.GatherScatterMode.PROMISE_IN_BOUNDS)
            return jnp.where(low, ga, gb)

        def fire_in(c, p):
            pltpu.async_copy(
                in_hbm.at[pl.ds(base + c * _R, _R)], ibufs[p], isems[p])

        def drain_in(c, p):
            pltpu.make_async_copy(
                in_hbm.at[pl.ds(base + c * _R, _R)], ibufs[p], isems[p]
            ).wait()

        def fire_out(c, p):
            pltpu.async_copy(
                obufs[p], out_hbm.at[pl.ds(base + c * _R, _R)], osems[p])

        def drain_out(c, p):
            pltpu.make_async_copy(
                obufs[p], out_hbm.at[pl.ds(base + c * _R, _R)], osems[p]
            ).wait()

        def compute(p):
            ib, ob = ibufs[p], obufs[p]

            @plsc.parallel_loop(0, _R, unroll=16)
            def _row(r):
                for q in range(_NQ):
                    a = ib[r, pl.ds(32 * q, 16)]
                    b = ib[r, pl.ds(32 * q + 16, 16)]
                    ob[r, pl.ds(16 * q, 16)] = compact(a, b)

        fire_in(0, 0)
        fire_in(1, 1)
        for c in range(_NCH):
            p = c % 2
            drain_in(c, p)
            if c >= 2:
                drain_out(c - 2, p)
            compute(p)
            fire_out(c, p)
            if c + 2 < _NCH:
                fire_in(c + 2, p)
        drain_out(_NCH - 2, 0)
        drain_out(_NCH - 1, 1)

    return sc_select


_sc_select = _make_sc_select()


def kernel(input_tensor):
    x = input_tensor.transpose(0, 2, 3, 1).reshape(_RTOT, _C)
    out2 = _sc_select(x)
    return out2.reshape(_B, _H, _W, _CO).transpose(0, 3, 1, 2)


# speedup vs baseline: 1.0359x; 1.0359x over previous
import functools

import jax
import jax.numpy as jnp
from jax import lax
from jax.experimental import pallas as pl
from jax.experimental.pallas import tpu as pltpu
from jax.experimental.pallas import tpu_sc as plsc

_B, _C, _H, _W = 64, 384, 24, 24
_CO = _C // 2
_RTOT = _B * _H * _W
_NW = 32
_RPW = _RTOT // _NW
_R = 64
_NCH = _RPW // _R
_NQ = _CO // 16


def _make_sc_select():
    mesh = plsc.VectorSubcoreMesh(core_axis_name="c", subcore_axis_name="s")

    @functools.partial(
        pl.kernel,
        mesh=mesh,
        out_type=jax.ShapeDtypeStruct((_RTOT, _CO), jnp.float32),
        scratch_types=[
            pltpu.VMEM((_R, _C), jnp.float32),
            pltpu.VMEM((_R, _C), jnp.float32),
            pltpu.VMEM((_R, _CO), jnp.float32),
            pltpu.VMEM((_R, _CO), jnp.float32),
            pltpu.SemaphoreType.DMA,
            pltpu.SemaphoreType.DMA,
            pltpu.SemaphoreType.DMA,
            pltpu.SemaphoreType.DMA,
        ],
    )
    def sc_select(in_hbm, out_hbm, ibuf0, ibuf1, obuf0, obuf1,
                  isem0, isem1, osem0, osem1):
        wid = lax.axis_index("s") * 2 + lax.axis_index("c")
        base = wid * _RPW
        ibufs = (ibuf0, ibuf1)
        obufs = (obuf0, obuf1)
        isems = (isem0, isem1)
        osems = (osem0, osem1)

        lane = lax.iota(jnp.int32, 16)
        evens = (2 * lane) % 16
        low = lane < 8
        dnums = lax.GatherDimensionNumbers(
            offset_dims=(), collapsed_slice_dims=(0,), start_index_map=(0,))

        def compact(a, b):
            ga = lax.gather(a, evens[:, None], dnums, slice_sizes=(1,),
                            mode=lax.GatherScatterMode.PROMISE_IN_BOUNDS)
            gb = lax.gather(b, evens[:, None], dnums, slice_sizes=(1,),
                            mode=lax.GatherScatterMode.PROMISE_IN_BOUNDS)
            return jnp.where(low, ga, gb)

        def fire_in(c, p):
            pltpu.async_copy(
                in_hbm.at[pl.ds(base + c * _R, _R)], ibufs[p], isems[p])

        def drain_in(c, p):
            pltpu.make_async_copy(
                in_hbm.at[pl.ds(base + c * _R, _R)], ibufs[p], isems[p]
            ).wait()

        def fire_out(c, p):
            pltpu.async_copy(
                obufs[p], out_hbm.at[pl.ds(base + c * _R, _R)], osems[p])

        def drain_out(c, p):
            pltpu.make_async_copy(
                obufs[p], out_hbm.at[pl.ds(base + c * _R, _R)], osems[p]
            ).wait()

        def compute(p):
            ib, ob = ibufs[p], obufs[p]

            @plsc.parallel_loop(0, _R, unroll=8)
            def _row(r):
                for q in range(_NQ):
                    a = ib[r, pl.ds(32 * q, 16)]
                    b = ib[r, pl.ds(32 * q + 16, 16)]
                    ob[r, pl.ds(16 * q, 16)] = compact(a, b)

        fire_in(0, 0)
        fire_in(1, 1)
        for c in range(_NCH):
            p = c % 2
            drain_in(c, p)
            if c >= 2:
                drain_out(c - 2, p)
            compute(p)
            fire_out(c, p)
            if c + 2 < _NCH:
                fire_in(c + 2, p)
        drain_out(_NCH - 2, 0)
        drain_out(_NCH - 1, 1)

    return sc_select


_sc_select = _make_sc_select()


def kernel(input_tensor):
    x = input_tensor.transpose(0, 2, 3, 1).reshape(_RTOT, _C)
    out2 = _sc_select(x)
    return out2.reshape(_B, _H, _W, _CO).transpose(0, 3, 1, 2)
